# baseline (speedup 1.0000x reference)
import functools

import jax
import jax.numpy as jnp
from jax import lax
from jax.experimental import pallas as pl
from jax.experimental.pallas import tpu as pltpu
from jax.experimental.pallas import tpu_sc as plsc

_THR = -20.0
_RANK_FIRST = 1996.0
_B = 16
_L = 2048
_CHUNK = 128
_NCH = _L // _CHUNK
_NV = _L // 16
_TBL = 30016
_IDENT = 0b11100100

_GATHER_DNUMS = lax.GatherDimensionNumbers(
    offset_dims=(), collapsed_slice_dims=(0,), start_index_map=(0,))


def _take16(x, idx):
    return lax.gather(x, idx[:, None], _GATHER_DNUMS, slice_sizes=(1,),
                      mode=lax.GatherScatterMode.PROMISE_IN_BOUNDS)


def _compose(tb, ta):
    res = jnp.zeros_like(ta)
    for i in range(4):
        v = (ta >> (2 * i)) & 3
        o = (tb >> (2 * v)) & 3
        res = res | (o << (2 * i))
    return res


@functools.partial(
    pl.kernel,
    mesh=plsc.VectorSubcoreMesh(core_axis_name="c", subcore_axis_name="s"),
    out_type=jax.ShapeDtypeStruct((_B, _L), jnp.int32),
    scratch_types=[
        pltpu.VMEM((_TBL,), jnp.float32),
        pltpu.VMEM((_L,), jnp.int32),
        pltpu.VMEM((_L,), jnp.int32),
        pltpu.VMEM((_L,), jnp.int32),
        pltpu.VMEM((_L,), jnp.float32),
        pltpu.VMEM((_L,), jnp.int32),
    ],
    compiler_params=pltpu.CompilerParams(needs_layout_passes=False),
)
def _sc_chunker(ids_t_hbm, m_t_hbm, m_hbm, tbl_hbm, out_hbm,
                tbl_v, ids_v, mt_v, m2_v, sums_v, o_v):
    wid = lax.axis_index("s") * 2 + lax.axis_index("c")

    @pl.when(wid < _B)
    def _():
        pltpu.sync_copy(tbl_hbm, tbl_v)
        pltpu.sync_copy(ids_t_hbm.at[wid], ids_v)
        pltpu.sync_copy(m_t_hbm.at[wid], mt_v)
        pltpu.sync_copy(m_hbm.at[wid], m2_v)
        lane = lax.iota(jnp.int32, 16)
        idx15 = jnp.full((16,), 15, jnp.int32)
        scat_base = lane * _CHUNK

        def step_sum(p, acc):
            ids16 = ids_v[pl.ds(p * 16, 16)]
            m16 = mt_v[pl.ds(p * 16, 16)]
            av = plsc.load_gather(tbl_v, [ids16])
            acc = acc + av * (m16 == 1).astype(jnp.float32)
            plsc.store_scatter(sums_v, [scat_base + p], acc)
            return acc

        totals = lax.fori_loop(0, _CHUNK, step_sum,
                               jnp.zeros((16,), jnp.float32))
        incl = totals
        for c in range(1, _NCH):
            b = _take16(incl, jnp.full((16,), c - 1, jnp.int32))
            t = _take16(totals, jnp.full((16,), c, jnp.int32))
            incl = jnp.where(lane == c, b + t, incl)
        carry = jnp.where(lane == 0, 0.0,
                          _take16(incl, jnp.maximum(lane - 1, 0)))

        def shifted(cur, prev, k):
            g1 = _take16(cur, jnp.maximum(lane - k, 0))
            g2 = _take16(prev, jnp.minimum(lane + 16 - k, 15))
            return jnp.where(lane >= k, g1, g2)

        def step_fsm(v, carry3):
            st_carry, prev_s, prev_m = carry3
            inner = sums_v[pl.ds(v * 16, 16)]
            cb = _take16(carry, jnp.zeros((16,), jnp.int32) + (v >> 3))
            full = inner + cb
            m_cur = m2_v[pl.ds(v * 16, 16)]
            c1 = (full - shifted(full, prev_s, 1)) < _THR
            c2 = (full - shifted(full, prev_s, 2)) < _THR
            c3 = (full - shifted(full, prev_s, 3)) < _THR
            mj = m_cur == 1
            mjm1 = shifted(m_cur, prev_m, 1) == 1
            nmj = ~mj
            g01 = (nmj | ~mjm1 | c1).astype(jnp.int32)
            g10 = (nmj | c2).astype(jnp.int32)
            g00 = (nmj | c3).astype(jnp.int32)
            T = (g00 | ((g01 | 2) << 2) | (g10 << 4) | ((g01 | 2) << 6))
            T = jnp.where((lane == 0) & (v == 0), _IDENT, T)
            P = T
            for d in (1, 2, 4, 8):
                sh = _take16(P, jnp.maximum(lane - d, 0))
                sh = jnp.where(lane >= d, sh, _IDENT)
                P = _compose(P, sh)
            st = (P >> (2 * st_carry)) & 3
            o_v[pl.ds(v * 16, 16)] = st & 1
            return (_take16(st, idx15), full, m_cur)

        lax.fori_loop(0, _NV, step_fsm,
                      (jnp.ones((16,), jnp.int32),
                       jnp.zeros((16,), jnp.float32),
                       jnp.zeros((16,), jnp.int32)))
        pltpu.sync_copy(o_v, out_hbm.at[wid])


def kernel(inp, padding_mask, regular_tokens_mask, token_ids):
    del inp, padding_mask
    tbl = -1.0 * jnp.log(jnp.arange(_TBL, dtype=jnp.float32) + _RANK_FIRST)
    tbl = lax.optimization_barrier(tbl)
    ids_t = token_ids.reshape(_B, _NCH, _CHUNK).transpose(0, 2, 1).reshape(_B, _L)
    m_t = regular_tokens_mask.reshape(_B, _NCH, _CHUNK).transpose(0, 2, 1).reshape(_B, _L)
    return _sc_chunker(ids_t, m_t, regular_tokens_mask, tbl)


# speedup vs baseline: 1.0874x; 1.0874x over previous
import functools

import jax
import jax.numpy as jnp
from jax import lax
from jax.experimental import pallas as pl
from jax.experimental.pallas import tpu as pltpu
from jax.experimental.pallas import tpu_sc as plsc

_THR = -20.0
_RANK_FIRST = 1996.0
_B = 16
_L = 2048
_CHUNK = 128
_NCH = _L // _CHUNK
_NV = _L // 16
_TBL = 30016
_IDENT = 0b11100100

_GATHER_DNUMS = lax.GatherDimensionNumbers(
    offset_dims=(), collapsed_slice_dims=(0,), start_index_map=(0,))


def _take16(x, idx):
    return lax.gather(x, idx[:, None], _GATHER_DNUMS, slice_sizes=(1,),
                      mode=lax.GatherScatterMode.PROMISE_IN_BOUNDS)


def _compose(tb, ta):
    res = jnp.zeros_like(ta)
    for i in range(4):
        v = (ta >> (2 * i)) & 3
        o = (tb >> (2 * v)) & 3
        res = res | (o << (2 * i))
    return res


@functools.partial(
    pl.kernel,
    mesh=plsc.VectorSubcoreMesh(core_axis_name="c", subcore_axis_name="s"),
    out_type=jax.ShapeDtypeStruct((_B, _L), jnp.int32),
    scratch_types=[
        pltpu.VMEM((_TBL,), jnp.float32),
        pltpu.VMEM((_L,), jnp.int32),
        pltpu.VMEM((_L,), jnp.int32),
        pltpu.VMEM((_L,), jnp.int32),
        pltpu.VMEM((_L,), jnp.float32),
        pltpu.VMEM((_L,), jnp.int32),
    ],
    compiler_params=pltpu.CompilerParams(needs_layout_passes=False),
)
def _sc_chunker(ids_t_hbm, m_t_hbm, m_hbm, tbl_hbm, out_hbm,
                tbl_v, ids_v, mt_v, m2_v, sums_v, o_v):
    wid = lax.axis_index("s") * 2 + lax.axis_index("c")

    @pl.when(wid < _B)
    def _():
        pltpu.sync_copy(tbl_hbm, tbl_v)
        pltpu.sync_copy(ids_t_hbm.at[wid], ids_v)
        pltpu.sync_copy(m_t_hbm.at[wid], mt_v)
        pltpu.sync_copy(m_hbm.at[wid], m2_v)
        lane = lax.iota(jnp.int32, 16)
        idx15 = jnp.full((16,), 15, jnp.int32)
        scat_base = lane * _CHUNK

        @plsc.parallel_loop(0, _CHUNK, carry=jnp.zeros((16,), jnp.float32),
                            unroll=4)
        def totals(p, acc):
            ids16 = ids_v[pl.ds(p * 16, 16)]
            m16 = mt_v[pl.ds(p * 16, 16)]
            av = plsc.load_gather(tbl_v, [ids16])
            acc = acc + av * (m16 == 1).astype(jnp.float32)
            plsc.store_scatter(sums_v, [scat_base + p], acc)
            return acc
        incl = totals
        for c in range(1, _NCH):
            b = _take16(incl, jnp.full((16,), c - 1, jnp.int32))
            t = _take16(totals, jnp.full((16,), c, jnp.int32))
            incl = jnp.where(lane == c, b + t, incl)
        carry = jnp.where(lane == 0, 0.0,
                          _take16(incl, jnp.maximum(lane - 1, 0)))

        def shifted(cur, prev, k):
            g1 = _take16(cur, jnp.maximum(lane - k, 0))
            g2 = _take16(prev, jnp.minimum(lane + 16 - k, 15))
            return jnp.where(lane >= k, g1, g2)

        @plsc.parallel_loop(0, _NV,
                            carry=(jnp.ones((16,), jnp.int32),
                                   jnp.zeros((16,), jnp.float32),
                                   jnp.zeros((16,), jnp.int32)),
                            unroll=4)
        def _fsm(v, carry3):
            st_carry, prev_s, prev_m = carry3
            inner = sums_v[pl.ds(v * 16, 16)]
            cb = _take16(carry, jnp.zeros((16,), jnp.int32) + (v >> 3))
            full = inner + cb
            m_cur = m2_v[pl.ds(v * 16, 16)]
            c1 = (full - shifted(full, prev_s, 1)) < _THR
            c2 = (full - shifted(full, prev_s, 2)) < _THR
            c3 = (full - shifted(full, prev_s, 3)) < _THR
            mj = m_cur == 1
            mjm1 = shifted(m_cur, prev_m, 1) == 1
            nmj = ~mj
            g01 = (nmj | ~mjm1 | c1).astype(jnp.int32)
            g10 = (nmj | c2).astype(jnp.int32)
            g00 = (nmj | c3).astype(jnp.int32)
            T = (g00 | ((g01 | 2) << 2) | (g10 << 4) | ((g01 | 2) << 6))
            T = jnp.where((lane == 0) & (v == 0), _IDENT, T)
            P = T
            for d in (1, 2, 4, 8):
                sh = _take16(P, jnp.maximum(lane - d, 0))
                sh = jnp.where(lane >= d, sh, _IDENT)
                P = _compose(P, sh)
            st = (P >> (2 * st_carry)) & 3
            o_v[pl.ds(v * 16, 16)] = st & 1
            return (_take16(st, idx15), full, m_cur)

        del _fsm
        pltpu.sync_copy(o_v, out_hbm.at[wid])


def kernel(inp, padding_mask, regular_tokens_mask, token_ids):
    del inp, padding_mask
    tbl = -1.0 * jnp.log(jnp.arange(_TBL, dtype=jnp.float32) + _RANK_FIRST)
    tbl = lax.optimization_barrier(tbl)
    ids_t = token_ids.reshape(_B, _NCH, _CHUNK).transpose(0, 2, 1).reshape(_B, _L)
    m_t = regular_tokens_mask.reshape(_B, _NCH, _CHUNK).transpose(0, 2, 1).reshape(_B, _L)
    return _sc_chunker(ids_t, m_t, regular_tokens_mask, tbl)


# speedup vs baseline: 1.0936x; 1.0058x over previous
import functools

import jax
import jax.numpy as jnp
from jax import lax
from jax.experimental import pallas as pl
from jax.experimental.pallas import tpu as pltpu
from jax.experimental.pallas import tpu_sc as plsc

_THR = -20.0
_RANK_FIRST = 1996.0
_B = 16
_L = 2048
_CHUNK = 128
_NCH = _L // _CHUNK
_NV = _L // 16
_TBL = 30016
_IDENT = 0b11100100

_GATHER_DNUMS = lax.GatherDimensionNumbers(
    offset_dims=(), collapsed_slice_dims=(0,), start_index_map=(0,))


def _take16(x, idx):
    return lax.gather(x, idx[:, None], _GATHER_DNUMS, slice_sizes=(1,),
                      mode=lax.GatherScatterMode.PROMISE_IN_BOUNDS)


def _compose(tb, ta):
    res = jnp.zeros_like(ta)
    for i in range(4):
        v = (ta >> (2 * i)) & 3
        o = (tb >> (2 * v)) & 3
        res = res | (o << (2 * i))
    return res


@functools.partial(
    pl.kernel,
    mesh=plsc.VectorSubcoreMesh(core_axis_name="c", subcore_axis_name="s"),
    out_type=jax.ShapeDtypeStruct((_B, _L), jnp.int32),
    scratch_types=[
        pltpu.VMEM((_TBL,), jnp.float32),
        pltpu.VMEM((_L,), jnp.int32),
        pltpu.VMEM((_L,), jnp.int32),
        pltpu.VMEM((_L,), jnp.int32),
        pltpu.VMEM((_L,), jnp.float32),
        pltpu.VMEM((_L,), jnp.int32),
    ],
    compiler_params=pltpu.CompilerParams(needs_layout_passes=False),
)
def _sc_chunker(ids_t_hbm, m_t_hbm, m_hbm, tbl_hbm, out_hbm,
                tbl_v, ids_v, mt_v, m2_v, sums_v, o_v):
    wid = lax.axis_index("s")

    @pl.when(lax.axis_index("c") == 0)
    def _():
        pltpu.sync_copy(tbl_hbm, tbl_v)
        pltpu.sync_copy(ids_t_hbm.at[wid], ids_v)
        pltpu.sync_copy(m_t_hbm.at[wid], mt_v)
        pltpu.sync_copy(m_hbm.at[wid], m2_v)
        lane = lax.iota(jnp.int32, 16)
        idx15 = jnp.full((16,), 15, jnp.int32)
        scat_base = lane * _CHUNK

        @plsc.parallel_loop(0, _CHUNK, carry=jnp.zeros((16,), jnp.float32),
                            unroll=4)
        def totals(p, acc):
            ids16 = ids_v[pl.ds(p * 16, 16)]
            m16 = mt_v[pl.ds(p * 16, 16)]
            av = plsc.load_gather(tbl_v, [ids16])
            acc = acc + av * (m16 == 1).astype(jnp.float32)
            plsc.store_scatter(sums_v, [scat_base + p], acc)
            return acc
        incl = totals
        for c in range(1, _NCH):
            b = _take16(incl, jnp.full((16,), c - 1, jnp.int32))
            t = _take16(totals, jnp.full((16,), c, jnp.int32))
            incl = jnp.where(lane == c, b + t, incl)
        carry = jnp.where(lane == 0, 0.0,
                          _take16(incl, jnp.maximum(lane - 1, 0)))

        def shifted(cur, prev, k):
            g1 = _take16(cur, jnp.maximum(lane - k, 0))
            g2 = _take16(prev, jnp.minimum(lane + 16 - k, 15))
            return jnp.where(lane >= k, g1, g2)

        @plsc.parallel_loop(0, _NV,
                            carry=(jnp.ones((16,), jnp.int32),
                                   jnp.zeros((16,), jnp.float32),
                                   jnp.zeros((16,), jnp.int32)),
                            unroll=4)
        def _fsm(v, carry3):
            st_carry, prev_s, prev_m = carry3
            inner = sums_v[pl.ds(v * 16, 16)]
            cb = _take16(carry, jnp.zeros((16,), jnp.int32) + (v >> 3))
            full = inner + cb
            m_cur = m2_v[pl.ds(v * 16, 16)]
            c1 = (full - shifted(full, prev_s, 1)) < _THR
            c2 = (full - shifted(full, prev_s, 2)) < _THR
            c3 = (full - shifted(full, prev_s, 3)) < _THR
            mj = m_cur == 1
            mjm1 = shifted(m_cur, prev_m, 1) == 1
            nmj = ~mj
            g01 = (nmj | ~mjm1 | c1).astype(jnp.int32)
            g10 = (nmj | c2).astype(jnp.int32)
            g00 = (nmj | c3).astype(jnp.int32)
            T = (g00 | ((g01 | 2) << 2) | (g10 << 4) | ((g01 | 2) << 6))
            T = jnp.where((lane == 0) & (v == 0), _IDENT, T)
            P = T
            for d in (1, 2, 4, 8):
                sh = _take16(P, jnp.maximum(lane - d, 0))
                sh = jnp.where(lane >= d, sh, _IDENT)
                P = _compose(P, sh)
            st = (P >> (2 * st_carry)) & 3
            o_v[pl.ds(v * 16, 16)] = st & 1
            return (_take16(st, idx15), full, m_cur)

        del _fsm
        pltpu.sync_copy(o_v, out_hbm.at[wid])


def kernel(inp, padding_mask, regular_tokens_mask, token_ids):
    del inp, padding_mask
    tbl = -1.0 * jnp.log(jnp.arange(_TBL, dtype=jnp.float32) + _RANK_FIRST)
    tbl = lax.optimization_barrier(tbl)
    ids_t = token_ids.reshape(_B, _NCH, _CHUNK).transpose(0, 2, 1).reshape(_B, _L)
    m_t = regular_tokens_mask.reshape(_B, _NCH, _CHUNK).transpose(0, 2, 1).reshape(_B, _L)
    return _sc_chunker(ids_t, m_t, regular_tokens_mask, tbl)


# speedup vs baseline: 1.2989x; 1.1877x over previous
import functools

import jax
import jax.numpy as jnp
from jax import lax
from jax.experimental import pallas as pl
from jax.experimental.pallas import tpu as pltpu
from jax.experimental.pallas import tpu_sc as plsc

_THR = -20.0
_RANK_FIRST = 1996.0
_B = 16
_L = 2048
_CHUNK = 128
_NCH = _L // _CHUNK
_COLS = _NCH * _B
_IDENT = 0b11100100


_GATHER_DNUMS = lax.GatherDimensionNumbers(
    offset_dims=(), collapsed_slice_dims=(0,), start_index_map=(0,))


def _take16(x, idx):
    return lax.gather(x, idx[:, None], _GATHER_DNUMS, slice_sizes=(1,),
                      mode=lax.GatherScatterMode.PROMISE_IN_BOUNDS)


def _compose(tb, ta):
    res = jnp.zeros_like(ta)
    for i in range(4):
        v = (ta >> (2 * i)) & 3
        o = (tb >> (2 * v)) & 3
        res = res | (o << (2 * i))
    return res


def _shift_pos(x, k, fill):
    wrap = x[_CHUNK - k:, :]
    wrap = jnp.concatenate(
        [jnp.full((k, _B), fill, x.dtype), wrap[:, :-_B]], axis=1)
    return jnp.concatenate([wrap, x[:_CHUNK - k, :]], axis=0)


def _tc_transitions(ids_ref, m_ref, t_out_ref):
    ids = ids_ref[...]
    m = m_ref[...]
    keep = m == 1
    a = (-1.0 * jnp.log(ids.astype(jnp.float32) + _RANK_FIRST)) * keep

    prev = a[0:1]
    rows = [prev]
    for p in range(1, _CHUNK):
        prev = prev + a[p:p + 1]
        rows.append(prev)
    inner = jnp.concatenate(rows, axis=0)
    tot = inner[_CHUNK - 1:_CHUNK]
    acc = jnp.zeros((1, _B), jnp.float32)
    pieces = [acc]
    for c in range(1, _NCH):
        acc = acc + tot[:, (c - 1) * _B:c * _B]
        pieces.append(acc)
    carry = jnp.concatenate(pieces, axis=1)
    sums = inner + carry

    c1 = (sums - _shift_pos(sums, 1, 0.0)) < _THR
    c2 = (sums - _shift_pos(sums, 2, 0.0)) < _THR
    c3 = (sums - _shift_pos(sums, 3, 0.0)) < _THR
    mj = keep
    mjm1 = _shift_pos(m, 1, 0) == 1
    nmj = ~mj
    g01 = (nmj | ~mjm1 | c1).astype(jnp.int32)
    g10 = (nmj | c2).astype(jnp.int32)
    g00 = (nmj | c3).astype(jnp.int32)
    T = (g00 | ((g01 | 2) << 2) | (g10 << 4) | ((g01 | 2) << 6))

    prow = jax.lax.broadcasted_iota(jnp.int32, T.shape, 0)
    pcol = jax.lax.broadcasted_iota(jnp.int32, T.shape, 1)
    at0 = (prow == 0) & (pcol < _B)
    t_out_ref[...] = jnp.where(at0, _IDENT, T)


@functools.partial(
    pl.kernel,
    mesh=plsc.VectorSubcoreMesh(core_axis_name="c", subcore_axis_name="s"),
    out_type=jax.ShapeDtypeStruct((_B, _L), jnp.int32),
    scratch_types=[
        pltpu.VMEM((_L,), jnp.int32),
        pltpu.VMEM((_L,), jnp.int32),
    ],
)
def _sc_scan(t_hbm, out_hbm, t_vmem, o_vmem):
    wid = lax.axis_index("s") * 2 + lax.axis_index("c")

    @pl.when(wid < _B)
    def _():
        pltpu.sync_copy(t_hbm.at[wid], t_vmem)
        lane = lax.iota(jnp.int32, 16)
        idx15 = jnp.full((16,), 15, jnp.int32)

        @plsc.parallel_loop(0, _L // 16, carry=jnp.ones((16,), jnp.int32),
                            unroll=4)
        def _scan(v, carry):
            P = t_vmem[pl.ds(v * 16, 16)]
            for d in (1, 2, 4, 8):
                sh = _take16(P, jnp.maximum(lane - d, 0))
                sh = jnp.where(lane >= d, sh, _IDENT)
                P = _compose(P, sh)
            st = (P >> (2 * carry)) & 3
            o_vmem[pl.ds(v * 16, 16)] = st & 1
            return _take16(st, idx15)

        del _scan
        pltpu.sync_copy(o_vmem, out_hbm.at[wid])


def kernel(inp, padding_mask, regular_tokens_mask, token_ids):
    del inp, padding_mask
    ids_t = token_ids.reshape(_B, _NCH, _CHUNK).transpose(2, 1, 0).reshape(_CHUNK, _COLS)
    m_t = regular_tokens_mask.reshape(_B, _NCH, _CHUNK).transpose(2, 1, 0).reshape(_CHUNK, _COLS)
    t_packed = pl.pallas_call(
        _tc_transitions,
        out_shape=jax.ShapeDtypeStruct((_CHUNK, _COLS), jnp.int32),
    )(ids_t, m_t)
    t_rows = t_packed.reshape(_CHUNK, _NCH, _B).transpose(2, 1, 0).reshape(_B, _L)
    return _sc_scan(t_rows)
